# baseline (speedup 1.0000x reference)
import functools

import jax
import jax.numpy as jnp
from jax import lax
from jax.experimental import pallas as pl
from jax.experimental.pallas import tpu as pltpu
from jax.experimental.pallas import tpu_sc as plsc

N_NODES = 10000
D_FEAT = 128
TOPK = 10
BR = 200

NC = 2
NS = 16
NW = NC * NS
CHUNK = 128
N_ACC = 10112
DUMP = 10000
E_EDGE = 160000
EDGE_CHUNKS32 = 80
EDGE_CHUNKS128 = 160
E_PAD_G = NS * EDGE_CHUNKS32 * CHUNK
NSK = N_NODES * TOPK
CORR_CHUNKS = 26
NSK_PAD = NW * CORR_CHUNKS * CHUNK
ROWS_OUT = N_ACC // NS


def _edge_agg_sc(d_feat, n_chunks, ck, serial=False):
    mesh = plsc.VectorSubcoreMesh(core_axis_name="c", subcore_axis_name="s")
    epw = n_chunks * ck

    def body(xm_s, eam_s, src_s, dst_s, xm_t, eam_t, src_t, dst_t, zeros_hbm,
             out_s, out_t,
             src_v0, dst_v0, rows_v0, src_v1, dst_v1, rows_v1,
             eam_v, acc, semL0, semL1, semG0, semG1, semE):
        cid = lax.axis_index("c")
        sid = lax.axis_index("s")
        zr = N_ACC // NS
        pltpu.sync_copy(zeros_hbm.at[pl.ds(sid * zr, zr)],
                        acc.at[pl.ds(sid * zr, zr)])
        plsc.subcore_barrier()
        bufs = ((src_v0, dst_v0, rows_v0, semL0, semG0),
                (src_v1, dst_v1, rows_v1, semL1, semG1))

        def run_graph(xm, eam, src, dst, out):
            def start_loads(b, k):
                src_v, dst_v, rows_v, semL, _ = b
                base = sid * epw + k * ck
                pltpu.async_copy(src.at[pl.ds(base, ck)], src_v, semL)
                pltpu.async_copy(dst.at[pl.ds(base, ck)], dst_v, semL)

            def start_eam(k):
                base = sid * epw + k * ck
                pltpu.async_copy(eam.at[pl.ds(base, ck)], eam_v, semE)

            def launch_gather(b):
                src_v, dst_v, rows_v, semL, semG = b
                pltpu.make_async_copy(src.at[pl.ds(0, ck)], src_v,
                                      semL).wait()
                pltpu.make_async_copy(dst.at[pl.ds(0, ck)], dst_v,
                                      semL).wait()
                pltpu.async_copy(xm.at[src_v], rows_v, semG)

            def finish(b, k):
                src_v, dst_v, rows_v, semL, semG = b
                pltpu.make_async_copy(xm.at[src_v], rows_v, semG).wait()
                pltpu.make_async_copy(eam.at[pl.ds(0, ck)], eam_v,
                                      semE).wait()

                def row_body(i, _):
                    for j in range(d_feat // 16):
                        sl = pl.ds(j * 16, 16)
                        rows_v[i, sl] = jnp.maximum(
                            rows_v[i, sl] + eam_v[i, sl], 0.0)
                    return ()
                lax.fori_loop(0, ck, row_body, (), unroll=4)

                @pl.when(k + 1 < n_chunks)
                def _():
                    start_eam(k + 1)
                pltpu.sync_copy(rows_v, acc.at[dst_v], add=True)

            def phase(this, nxt, k):
                @pl.when(k + 1 < n_chunks)
                def _():
                    launch_gather(nxt)
                finish(this, k)

                @pl.when(k + 2 < n_chunks)
                def _():
                    start_loads(this, k + 2)

            if serial:
                def chunk_body(k, _):
                    base = sid * epw + k * ck
                    pltpu.sync_copy(src.at[pl.ds(base, ck)], src_v0)
                    pltpu.sync_copy(dst.at[pl.ds(base, ck)], dst_v0)
                    gth = pltpu.async_copy(xm.at[src_v0], rows_v0, semG0)
                    pltpu.sync_copy(eam.at[pl.ds(base, ck)], eam_v)
                    gth.wait()

                    def row_body(i, _):
                        for j in range(d_feat // 16):
                            sl = pl.ds(j * 16, 16)
                            rows_v0[i, sl] = jnp.maximum(
                                rows_v0[i, sl] + eam_v[i, sl], 0.0)
                        return ()
                    lax.fori_loop(0, ck, row_body, (), unroll=False)
                    pltpu.sync_copy(rows_v0, acc.at[dst_v0], add=True)
                    return ()
                lax.fori_loop(0, n_chunks, chunk_body, (), unroll=False)
            else:
                start_loads(bufs[0], 0)
                start_loads(bufs[1], 1)
                start_eam(0)
                launch_gather(bufs[0])

                def pair_body(g2, _):
                    phase(bufs[0], bufs[1], 2 * g2)
                    phase(bufs[1], bufs[0], 2 * g2 + 1)
                    return ()
                lax.fori_loop(0, n_chunks // 2, pair_body, (), unroll=False)
            plsc.subcore_barrier()
            pltpu.sync_copy(acc.at[pl.ds(sid * ROWS_OUT, ROWS_OUT)],
                            out.at[pl.ds(sid * ROWS_OUT, ROWS_OUT)])

        @pl.when(cid == 0)
        def _():
            run_graph(xm_s, eam_s, src_s, dst_s, out_s)

        @pl.when(cid == 1)
        def _():
            run_graph(xm_t, eam_t, src_t, dst_t, out_t)

    return pl.kernel(
        body,
        out_type=[
            jax.ShapeDtypeStruct((N_ACC, d_feat), jnp.float32),
            jax.ShapeDtypeStruct((N_ACC, d_feat), jnp.float32),
        ],
        mesh=mesh,
        compiler_params=pltpu.CompilerParams(use_tc_tiling_on_sc=False),
        scratch_types=[
            pltpu.VMEM((ck,), jnp.int32),
            pltpu.VMEM((ck,), jnp.int32),
            pltpu.VMEM((ck, d_feat), jnp.float32),
            pltpu.VMEM((ck,), jnp.int32),
            pltpu.VMEM((ck,), jnp.int32),
            pltpu.VMEM((ck, d_feat), jnp.float32),
            pltpu.VMEM((ck, d_feat), jnp.float32),
            pltpu.VMEM_SHARED((N_ACC, d_feat), jnp.float32),
            pltpu.SemaphoreType.DMA,
            pltpu.SemaphoreType.DMA,
            pltpu.SemaphoreType.DMA,
            pltpu.SemaphoreType.DMA,
            pltpu.SemaphoreType.DMA,
        ],
    )


def _scatter_sum_sc(d_feat, n_chunks):
    mesh = plsc.VectorSubcoreMesh(core_axis_name="c", subcore_axis_name="s")
    rpw = n_chunks * CHUNK

    def body(vals, idx, zeros_hbm, out, vals_v0, idx_v0, vals_v1, idx_v1,
             acc, semL0, semL1):
        cid = lax.axis_index("c")
        sid = lax.axis_index("s")
        wid = cid * NS + sid
        zr = N_ACC // NS
        pltpu.sync_copy(zeros_hbm.at[pl.ds(sid * zr, zr)],
                        acc.at[pl.ds(sid * zr, zr)])
        plsc.subcore_barrier()
        bufs = ((vals_v0, idx_v0, semL0), (vals_v1, idx_v1, semL1))

        def start_loads(b, k):
            vals_v, idx_v, semL = b
            base = wid * rpw + k * CHUNK
            pltpu.async_copy(idx.at[pl.ds(base, CHUNK)], idx_v, semL)
            pltpu.async_copy(vals.at[pl.ds(base, CHUNK)], vals_v, semL)

        def phase(this, nxt, k):
            vals_v, idx_v, semL = this

            @pl.when(k + 1 < n_chunks)
            def _():
                start_loads(nxt, k + 1)
            pltpu.make_async_copy(idx.at[pl.ds(0, CHUNK)], idx_v,
                                  semL).wait()
            pltpu.make_async_copy(vals.at[pl.ds(0, CHUNK)], vals_v,
                                  semL).wait()
            pltpu.sync_copy(vals_v, acc.at[idx_v], add=True)

        start_loads(bufs[0], 0)

        def pair_body(g2, _):
            phase(bufs[0], bufs[1], 2 * g2)
            phase(bufs[1], bufs[0], 2 * g2 + 1)
            return ()
        lax.fori_loop(0, n_chunks // 2, pair_body, (), unroll=False)
        plsc.subcore_barrier()
        pltpu.sync_copy(acc.at[pl.ds(sid * ROWS_OUT, ROWS_OUT)],
                        out.at[cid, pl.ds(sid * ROWS_OUT, ROWS_OUT)])

    return pl.kernel(
        body,
        out_type=jax.ShapeDtypeStruct((NC, N_ACC, d_feat), jnp.float32),
        mesh=mesh,
        compiler_params=pltpu.CompilerParams(use_tc_tiling_on_sc=False),
        scratch_types=[
            pltpu.VMEM((CHUNK, d_feat), jnp.float32),
            pltpu.VMEM((CHUNK,), jnp.int32),
            pltpu.VMEM((CHUNK, d_feat), jnp.float32),
            pltpu.VMEM((CHUNK,), jnp.int32),
            pltpu.VMEM_SHARED((N_ACC, d_feat), jnp.float32),
            pltpu.SemaphoreType.DMA,
            pltpu.SemaphoreType.DMA,
        ],
    )


def _gather_rows_sc(d_feat, n_chunks, n_rows_out):
    mesh = plsc.VectorSubcoreMesh(core_axis_name="c", subcore_axis_name="s")
    rpw = n_chunks * CHUNK

    def body(table, idx, out, idx_v0, rows_v0, idx_v1, rows_v1,
             semL0, semL1, semG0, semG1):
        cid = lax.axis_index("c")
        sid = lax.axis_index("s")
        wid = cid * NS + sid
        bufs = ((idx_v0, rows_v0, semL0, semG0),
                (idx_v1, rows_v1, semL1, semG1))

        def start_load(b, k):
            idx_v, _, semL, _ = b
            base = wid * rpw + k * CHUNK
            pltpu.async_copy(idx.at[pl.ds(base, CHUNK)], idx_v, semL)

        def launch_gather(b):
            idx_v, rows_v, semL, semG = b
            pltpu.make_async_copy(idx.at[pl.ds(0, CHUNK)], idx_v,
                                  semL).wait()
            pltpu.async_copy(table.at[idx_v], rows_v, semG)

        def phase(this, nxt, k):
            idx_v, rows_v, semL, semG = this

            @pl.when(k + 1 < n_chunks)
            def _():
                launch_gather(nxt)
            pltpu.make_async_copy(table.at[idx_v], rows_v, semG).wait()
            base = wid * rpw + k * CHUNK
            pltpu.sync_copy(rows_v, out.at[pl.ds(base, CHUNK)])

            @pl.when(k + 2 < n_chunks)
            def _():
                start_load(this, k + 2)

        start_load(bufs[0], 0)
        start_load(bufs[1], 1)
        launch_gather(bufs[0])

        def pair_body(g2, _):
            phase(bufs[0], bufs[1], 2 * g2)
            phase(bufs[1], bufs[0], 2 * g2 + 1)
            return ()
        lax.fori_loop(0, n_chunks // 2, pair_body, (), unroll=False)

    return pl.kernel(
        body,
        out_type=jax.ShapeDtypeStruct((n_rows_out, d_feat), jnp.float32),
        mesh=mesh,
        compiler_params=pltpu.CompilerParams(use_tc_tiling_on_sc=False),
        scratch_types=[
            pltpu.VMEM((CHUNK,), jnp.int32),
            pltpu.VMEM((CHUNK, d_feat), jnp.float32),
            pltpu.VMEM((CHUNK,), jnp.int32),
            pltpu.VMEM((CHUNK, d_feat), jnp.float32),
            pltpu.SemaphoreType.DMA,
            pltpu.SemaphoreType.DMA,
            pltpu.SemaphoreType.DMA,
            pltpu.SemaphoreType.DMA,
        ],
    )


def _simtopk_body(hs_ref, ht_ref, val_ref, idx_ref):
    sim = lax.dot_general(
        hs_ref[...], ht_ref[...],
        dimension_numbers=(((1,), (1,)), ((), ())),
        preferred_element_type=jnp.float32,
    )
    colid = lax.broadcasted_iota(jnp.int32, sim.shape, 1)
    neg_inf = jnp.float32(-jnp.inf)
    big = jnp.int32(2**30)
    m = jnp.max(sim, axis=1, keepdims=True)
    c = jnp.min(jnp.where(sim == m, colid, big), axis=1, keepdims=True)
    vals = [m]
    idxs = [c]
    for _ in range(TOPK - 1):
        avail = (sim < m) | ((sim == m) & (colid > c))
        m2 = jnp.max(jnp.where(avail, sim, neg_inf), axis=1, keepdims=True)
        c2 = jnp.min(jnp.where((sim == m2) & ((m2 < m) | (colid > c)),
                               colid, big), axis=1, keepdims=True)
        m, c = m2, c2
        vals.append(m)
        idxs.append(c)
    val_ref[...] = jnp.concatenate(vals, axis=1)
    idx_ref[...] = jnp.concatenate(idxs, axis=1)


@jax.jit
def _simtopk(h_s, h_t):
    n_s = h_s.shape[0]
    grid = n_s // BR
    return pl.pallas_call(
        _simtopk_body,
        grid=(grid,),
        in_specs=[
            pl.BlockSpec((BR, D_FEAT), lambda i: (i, 0)),
            pl.BlockSpec((h_t.shape[0], D_FEAT), lambda i: (0, 0)),
        ],
        out_specs=[
            pl.BlockSpec((BR, TOPK), lambda i: (i, 0)),
            pl.BlockSpec((BR, TOPK), lambda i: (i, 0)),
        ],
        out_shape=[
            jax.ShapeDtypeStruct((n_s, TOPK), jnp.float32),
            jax.ShapeDtypeStruct((n_s, TOPK), jnp.int32),
        ],
    )(h_s, h_t)


def _pad_edges(edge_index, edge_attr):
    e = edge_index.shape[1]
    pad = E_PAD_G - e
    src = jnp.pad(edge_index[0], (0, pad))
    dst = jnp.pad(edge_index[1], (0, pad), constant_values=DUMP)
    ea = jnp.pad(edge_attr, ((0, pad), (0, 0)))
    return src, dst, ea


def kernel(x_s, edge_index_s, edge_attr_s, batch_s, x_t, edge_index_t,
           edge_attr_t, batch_t, W1r, W1m, W1e, b1, W2r, W2m, W2e, b2,
           M1, mb1, M2, mb2):
    n_s = x_s.shape[0]
    n_t = x_t.shape[0]
    src_s, dst_s, ea_s = _pad_edges(edge_index_s, edge_attr_s)
    src_t, dst_t, ea_t = _pad_edges(edge_index_t, edge_attr_t)
    zeros128 = jnp.zeros((N_ACC, 128), jnp.float32)
    zeros32 = jnp.zeros((N_ACC, 32), jnp.float32)

    edge_agg128 = _edge_agg_sc(128, 79, 128, serial=True)
    edge_agg32 = _edge_agg_sc(32, EDGE_CHUNKS32, 128)
    scatter32 = _scatter_sum_sc(32, CORR_CHUNKS)
    gather32 = _gather_rows_sc(32, CORR_CHUNKS, NSK_PAD)

    agg_s, agg_t = edge_agg128(x_s @ W1m, ea_s @ W1e, src_s, dst_s,
                               x_t @ W1m, ea_t @ W1e, src_t, dst_t, zeros128)
    h_s = jax.nn.relu(x_s @ W1r + agg_s[:N_NODES] + b1)
    h_t = jax.nn.relu(x_t @ W1r + agg_t[:N_NODES] + b1)

    S_hat, s_idx = _simtopk(h_s, h_t)
    S_0 = jax.nn.softmax(S_hat, axis=-1)

    rng = jax.random.key(12345)
    eam2_s = ea_s @ W2e
    eam2_t = ea_t @ W2e
    flat_idx = s_idx.reshape(-1)
    idx_pad = jnp.pad(flat_idx, (0, NSK_PAD - NSK), constant_values=DUMP)
    idx_pad0 = jnp.pad(flat_idx, (0, NSK_PAD - NSK))
    for step in range(2):
        S = jax.nn.softmax(S_hat, axis=-1)
        r_s = jax.random.normal(jax.random.fold_in(rng, step), (n_s, 32),
                                jnp.float32)
        tmp = (r_s[:, None, :] * S[:, :, None]).reshape(-1, 32)
        tmp = jnp.pad(tmp, ((0, NSK_PAD - NSK), (0, 0)))
        parts = scatter32(tmp, idx_pad, zeros32)
        r_t = parts[0, :N_NODES] + parts[1, :N_NODES]

        agg2_s, agg2_t = edge_agg32(r_s @ W2m, eam2_s, src_s, dst_s,
                                    r_t @ W2m, eam2_t, src_t, dst_t, zeros32)
        o_s = jax.nn.relu(r_s @ W2r + agg2_s[:N_NODES] + b2)
        o_t = jax.nn.relu(r_t @ W2r + agg2_t[:N_NODES] + b2)
        ot_g = gather32(o_t, idx_pad0)[:NSK].reshape(n_s, TOPK, 32)
        D = o_s[:, None, :] - ot_g
        upd = (jax.nn.relu(D @ M1 + mb1) @ M2 + mb2)[..., 0]
        S_hat = S_hat + upd
    S_L = jax.nn.softmax(S_hat, axis=-1)
    return (S_0[None], S_L[None], s_idx[None])


# speedup vs baseline: 1.5242x; 1.5242x over previous
import functools

import jax
import jax.numpy as jnp
from jax import lax
from jax.experimental import pallas as pl
from jax.experimental.pallas import tpu as pltpu
from jax.experimental.pallas import tpu_sc as plsc

N_NODES = 10000
D_FEAT = 128
TOPK = 10
BR = 200

NC = 2
NS = 16
NW = NC * NS
CHUNK = 128
N_ACC = 10112
DUMP = 10000
E_EDGE = 160000
EDGE_CHUNKS32 = 80
EDGE_CHUNKS128 = 160
E_PAD_G = NS * EDGE_CHUNKS32 * CHUNK
NSK = N_NODES * TOPK
CORR_CHUNKS = 26
NSK_PAD = NW * CORR_CHUNKS * CHUNK
ROWS_OUT = N_ACC // NS


def _edge_agg_sc(d_feat, n_chunks, ck, serial=False):
    mesh = plsc.VectorSubcoreMesh(core_axis_name="c", subcore_axis_name="s")
    epw = n_chunks * ck

    def body(xm_s, eam_s, src_s, dst_s, xm_t, eam_t, src_t, dst_t, zeros_hbm,
             out_s, out_t,
             src_v0, dst_v0, rows_v0, src_v1, dst_v1, rows_v1,
             eam_v, acc, semL0, semL1, semG0, semG1, semE):
        cid = lax.axis_index("c")
        sid = lax.axis_index("s")
        zr = N_ACC // NS
        pltpu.sync_copy(zeros_hbm.at[pl.ds(sid * zr, zr)],
                        acc.at[pl.ds(sid * zr, zr)])
        plsc.subcore_barrier()
        bufs = ((src_v0, dst_v0, rows_v0, semL0, semG0),
                (src_v1, dst_v1, rows_v1, semL1, semG1))

        def run_graph(xm, eam, src, dst, out):
            def start_loads(b, k):
                src_v, dst_v, rows_v, semL, _ = b
                base = sid * epw + k * ck
                pltpu.async_copy(src.at[pl.ds(base, ck)], src_v, semL)
                pltpu.async_copy(dst.at[pl.ds(base, ck)], dst_v, semL)

            def start_eam(k):
                base = sid * epw + k * ck
                pltpu.async_copy(eam.at[pl.ds(base, ck)], eam_v, semE)

            def launch_gather(b):
                src_v, dst_v, rows_v, semL, semG = b
                pltpu.make_async_copy(src.at[pl.ds(0, ck)], src_v,
                                      semL).wait()
                pltpu.make_async_copy(dst.at[pl.ds(0, ck)], dst_v,
                                      semL).wait()
                pltpu.async_copy(xm.at[src_v], rows_v, semG)

            def finish(b, k):
                src_v, dst_v, rows_v, semL, semG = b
                pltpu.make_async_copy(xm.at[src_v], rows_v, semG).wait()
                pltpu.make_async_copy(eam.at[pl.ds(0, ck)], eam_v,
                                      semE).wait()

                def row_body(i, _):
                    for j in range(d_feat // 16):
                        sl = pl.ds(j * 16, 16)
                        rows_v[i, sl] = jnp.maximum(
                            rows_v[i, sl] + eam_v[i, sl], 0.0)
                    return ()
                lax.fori_loop(0, ck, row_body, (), unroll=4)

                @pl.when(k + 1 < n_chunks)
                def _():
                    start_eam(k + 1)
                pltpu.sync_copy(rows_v, acc.at[dst_v], add=True)

            def phase(this, nxt, k):
                @pl.when(k + 1 < n_chunks)
                def _():
                    launch_gather(nxt)
                finish(this, k)

                @pl.when(k + 2 < n_chunks)
                def _():
                    start_loads(this, k + 2)

            if serial:
                def chunk_body(k, _):
                    base = sid * epw + k * ck
                    pltpu.sync_copy(src.at[pl.ds(base, ck)], src_v0)
                    pltpu.sync_copy(dst.at[pl.ds(base, ck)], dst_v0)
                    gth = pltpu.async_copy(xm.at[src_v0], rows_v0, semG0)
                    pltpu.sync_copy(eam.at[pl.ds(base, ck)], eam_v)
                    gth.wait()

                    def row_body(i, _):
                        for j in range(d_feat // 16):
                            sl = pl.ds(j * 16, 16)
                            rows_v0[i, sl] = jnp.maximum(
                                rows_v0[i, sl] + eam_v[i, sl], 0.0)
                        return ()
                    lax.fori_loop(0, ck, row_body, (), unroll=False)
                    pltpu.sync_copy(rows_v0, acc.at[dst_v0], add=True)
                    return ()
                lax.fori_loop(0, n_chunks, chunk_body, (), unroll=False)
            else:
                start_loads(bufs[0], 0)
                start_loads(bufs[1], 1)
                start_eam(0)
                launch_gather(bufs[0])

                def pair_body(g2, _):
                    phase(bufs[0], bufs[1], 2 * g2)
                    phase(bufs[1], bufs[0], 2 * g2 + 1)
                    return ()
                lax.fori_loop(0, n_chunks // 2, pair_body, (), unroll=False)
            plsc.subcore_barrier()
            pltpu.sync_copy(acc.at[pl.ds(sid * ROWS_OUT, ROWS_OUT)],
                            out.at[pl.ds(sid * ROWS_OUT, ROWS_OUT)])

        @pl.when(cid == 0)
        def _():
            run_graph(xm_s, eam_s, src_s, dst_s, out_s)

        @pl.when(cid == 1)
        def _():
            run_graph(xm_t, eam_t, src_t, dst_t, out_t)

    return pl.kernel(
        body,
        out_type=[
            jax.ShapeDtypeStruct((N_ACC, d_feat), jnp.float32),
            jax.ShapeDtypeStruct((N_ACC, d_feat), jnp.float32),
        ],
        mesh=mesh,
        compiler_params=pltpu.CompilerParams(use_tc_tiling_on_sc=False),
        scratch_types=[
            pltpu.VMEM((ck,), jnp.int32),
            pltpu.VMEM((ck,), jnp.int32),
            pltpu.VMEM((ck, d_feat), jnp.float32),
            pltpu.VMEM((ck,), jnp.int32),
            pltpu.VMEM((ck,), jnp.int32),
            pltpu.VMEM((ck, d_feat), jnp.float32),
            pltpu.VMEM((ck, d_feat), jnp.float32),
            pltpu.VMEM_SHARED((N_ACC, d_feat), jnp.float32),
            pltpu.SemaphoreType.DMA,
            pltpu.SemaphoreType.DMA,
            pltpu.SemaphoreType.DMA,
            pltpu.SemaphoreType.DMA,
            pltpu.SemaphoreType.DMA,
        ],
    )


def _scatter_sum_sc(d_feat, n_chunks):
    mesh = plsc.VectorSubcoreMesh(core_axis_name="c", subcore_axis_name="s")
    rpw = n_chunks * CHUNK

    def body(vals, idx, zeros_hbm, out, vals_v0, idx_v0, vals_v1, idx_v1,
             acc, semL0, semL1):
        cid = lax.axis_index("c")
        sid = lax.axis_index("s")
        wid = cid * NS + sid
        zr = N_ACC // NS
        pltpu.sync_copy(zeros_hbm.at[pl.ds(sid * zr, zr)],
                        acc.at[pl.ds(sid * zr, zr)])
        plsc.subcore_barrier()
        bufs = ((vals_v0, idx_v0, semL0), (vals_v1, idx_v1, semL1))

        def start_loads(b, k):
            vals_v, idx_v, semL = b
            base = wid * rpw + k * CHUNK
            pltpu.async_copy(idx.at[pl.ds(base, CHUNK)], idx_v, semL)
            pltpu.async_copy(vals.at[pl.ds(base, CHUNK)], vals_v, semL)

        def phase(this, nxt, k):
            vals_v, idx_v, semL = this

            @pl.when(k + 1 < n_chunks)
            def _():
                start_loads(nxt, k + 1)
            pltpu.make_async_copy(idx.at[pl.ds(0, CHUNK)], idx_v,
                                  semL).wait()
            pltpu.make_async_copy(vals.at[pl.ds(0, CHUNK)], vals_v,
                                  semL).wait()
            pltpu.sync_copy(vals_v, acc.at[idx_v], add=True)

        start_loads(bufs[0], 0)

        def pair_body(g2, _):
            phase(bufs[0], bufs[1], 2 * g2)
            phase(bufs[1], bufs[0], 2 * g2 + 1)
            return ()
        lax.fori_loop(0, n_chunks // 2, pair_body, (), unroll=False)
        plsc.subcore_barrier()
        pltpu.sync_copy(acc.at[pl.ds(sid * ROWS_OUT, ROWS_OUT)],
                        out.at[cid, pl.ds(sid * ROWS_OUT, ROWS_OUT)])

    return pl.kernel(
        body,
        out_type=jax.ShapeDtypeStruct((NC, N_ACC, d_feat), jnp.float32),
        mesh=mesh,
        compiler_params=pltpu.CompilerParams(use_tc_tiling_on_sc=False),
        scratch_types=[
            pltpu.VMEM((CHUNK, d_feat), jnp.float32),
            pltpu.VMEM((CHUNK,), jnp.int32),
            pltpu.VMEM((CHUNK, d_feat), jnp.float32),
            pltpu.VMEM((CHUNK,), jnp.int32),
            pltpu.VMEM_SHARED((N_ACC, d_feat), jnp.float32),
            pltpu.SemaphoreType.DMA,
            pltpu.SemaphoreType.DMA,
        ],
    )


def _gather_rows_sc(d_feat, n_chunks, n_rows_out):
    mesh = plsc.VectorSubcoreMesh(core_axis_name="c", subcore_axis_name="s")
    rpw = n_chunks * CHUNK

    def body(table, idx, out, idx_v0, rows_v0, idx_v1, rows_v1,
             semL0, semL1, semG0, semG1):
        cid = lax.axis_index("c")
        sid = lax.axis_index("s")
        wid = cid * NS + sid
        bufs = ((idx_v0, rows_v0, semL0, semG0),
                (idx_v1, rows_v1, semL1, semG1))

        def start_load(b, k):
            idx_v, _, semL, _ = b
            base = wid * rpw + k * CHUNK
            pltpu.async_copy(idx.at[pl.ds(base, CHUNK)], idx_v, semL)

        def launch_gather(b):
            idx_v, rows_v, semL, semG = b
            pltpu.make_async_copy(idx.at[pl.ds(0, CHUNK)], idx_v,
                                  semL).wait()
            pltpu.async_copy(table.at[idx_v], rows_v, semG)

        def phase(this, nxt, k):
            idx_v, rows_v, semL, semG = this

            @pl.when(k + 1 < n_chunks)
            def _():
                launch_gather(nxt)
            pltpu.make_async_copy(table.at[idx_v], rows_v, semG).wait()
            base = wid * rpw + k * CHUNK
            pltpu.sync_copy(rows_v, out.at[pl.ds(base, CHUNK)])

            @pl.when(k + 2 < n_chunks)
            def _():
                start_load(this, k + 2)

        start_load(bufs[0], 0)
        start_load(bufs[1], 1)
        launch_gather(bufs[0])

        def pair_body(g2, _):
            phase(bufs[0], bufs[1], 2 * g2)
            phase(bufs[1], bufs[0], 2 * g2 + 1)
            return ()
        lax.fori_loop(0, n_chunks // 2, pair_body, (), unroll=False)

    return pl.kernel(
        body,
        out_type=jax.ShapeDtypeStruct((n_rows_out, d_feat), jnp.float32),
        mesh=mesh,
        compiler_params=pltpu.CompilerParams(use_tc_tiling_on_sc=False),
        scratch_types=[
            pltpu.VMEM((CHUNK,), jnp.int32),
            pltpu.VMEM((CHUNK, d_feat), jnp.float32),
            pltpu.VMEM((CHUNK,), jnp.int32),
            pltpu.VMEM((CHUNK, d_feat), jnp.float32),
            pltpu.SemaphoreType.DMA,
            pltpu.SemaphoreType.DMA,
            pltpu.SemaphoreType.DMA,
            pltpu.SemaphoreType.DMA,
        ],
    )


def _simtopk_body(hs_ref, ht_ref, val_ref, idx_ref):
    sim = lax.dot_general(
        hs_ref[...], ht_ref[...],
        dimension_numbers=(((1,), (1,)), ((), ())),
        preferred_element_type=jnp.float32,
    )
    colid = lax.broadcasted_iota(jnp.int32, sim.shape, 1).astype(jnp.float32)
    neg_inf = jnp.float32(-jnp.inf)
    big = jnp.float32(3e7)
    vals = []
    idxs = []
    for _ in range(TOPK):
        m = jnp.max(sim, axis=1, keepdims=True)
        amax = jnp.min(jnp.where(sim >= m, colid, big), axis=1,
                       keepdims=True)
        vals.append(m)
        idxs.append(amax)
        sim = jnp.where(colid == amax, neg_inf, sim)
    val_ref[...] = jnp.concatenate(vals, axis=1)
    idx_ref[...] = jnp.concatenate(
        [i.astype(jnp.int32) for i in idxs], axis=1)


@jax.jit
def _simtopk(h_s, h_t):
    n_s = h_s.shape[0]
    grid = n_s // BR
    return pl.pallas_call(
        _simtopk_body,
        grid=(grid,),
        in_specs=[
            pl.BlockSpec((BR, D_FEAT), lambda i: (i, 0)),
            pl.BlockSpec((h_t.shape[0], D_FEAT), lambda i: (0, 0)),
        ],
        out_specs=[
            pl.BlockSpec((BR, TOPK), lambda i: (i, 0)),
            pl.BlockSpec((BR, TOPK), lambda i: (i, 0)),
        ],
        out_shape=[
            jax.ShapeDtypeStruct((n_s, TOPK), jnp.float32),
            jax.ShapeDtypeStruct((n_s, TOPK), jnp.int32),
        ],
    )(h_s, h_t)


def _pad_edges(edge_index, edge_attr):
    e = edge_index.shape[1]
    pad = E_PAD_G - e
    src = jnp.pad(edge_index[0], (0, pad))
    dst = jnp.pad(edge_index[1], (0, pad), constant_values=DUMP)
    ea = jnp.pad(edge_attr, ((0, pad), (0, 0)))
    return src, dst, ea


def kernel(x_s, edge_index_s, edge_attr_s, batch_s, x_t, edge_index_t,
           edge_attr_t, batch_t, W1r, W1m, W1e, b1, W2r, W2m, W2e, b2,
           M1, mb1, M2, mb2):
    n_s = x_s.shape[0]
    n_t = x_t.shape[0]
    src_s, dst_s, ea_s = _pad_edges(edge_index_s, edge_attr_s)
    src_t, dst_t, ea_t = _pad_edges(edge_index_t, edge_attr_t)
    zeros128 = jnp.zeros((N_ACC, 128), jnp.float32)
    zeros32 = jnp.zeros((N_ACC, 32), jnp.float32)

    edge_agg128 = _edge_agg_sc(128, 79, 128, serial=True)
    edge_agg32 = _edge_agg_sc(32, EDGE_CHUNKS32, 128)
    scatter32 = _scatter_sum_sc(32, CORR_CHUNKS)
    gather32 = _gather_rows_sc(32, CORR_CHUNKS, NSK_PAD)

    agg_s, agg_t = edge_agg128(x_s @ W1m, ea_s @ W1e, src_s, dst_s,
                               x_t @ W1m, ea_t @ W1e, src_t, dst_t, zeros128)
    h_s = jax.nn.relu(x_s @ W1r + agg_s[:N_NODES] + b1)
    h_t = jax.nn.relu(x_t @ W1r + agg_t[:N_NODES] + b1)

    S_hat, s_idx = _simtopk(h_s, h_t)
    S_0 = jax.nn.softmax(S_hat, axis=-1)

    rng = jax.random.key(12345)
    eam2_s = ea_s @ W2e
    eam2_t = ea_t @ W2e
    flat_idx = s_idx.reshape(-1)
    idx_pad = jnp.pad(flat_idx, (0, NSK_PAD - NSK), constant_values=DUMP)
    idx_pad0 = jnp.pad(flat_idx, (0, NSK_PAD - NSK))
    for step in range(2):
        S = jax.nn.softmax(S_hat, axis=-1)
        r_s = jax.random.normal(jax.random.fold_in(rng, step), (n_s, 32),
                                jnp.float32)
        tmp = (r_s[:, None, :] * S[:, :, None]).reshape(-1, 32)
        tmp = jnp.pad(tmp, ((0, NSK_PAD - NSK), (0, 0)))
        parts = scatter32(tmp, idx_pad, zeros32)
        r_t = parts[0, :N_NODES] + parts[1, :N_NODES]

        agg2_s, agg2_t = edge_agg32(r_s @ W2m, eam2_s, src_s, dst_s,
                                    r_t @ W2m, eam2_t, src_t, dst_t, zeros32)
        o_s = jax.nn.relu(r_s @ W2r + agg2_s[:N_NODES] + b2)
        o_t = jax.nn.relu(r_t @ W2r + agg2_t[:N_NODES] + b2)
        ot_g = gather32(o_t, idx_pad0)[:NSK].reshape(n_s, TOPK, 32)
        D = o_s[:, None, :] - ot_g
        upd = (jax.nn.relu(D @ M1 + mb1) @ M2 + mb2)[..., 0]
        S_hat = S_hat + upd
    S_L = jax.nn.softmax(S_hat, axis=-1)
    return (S_0[None], S_L[None], s_idx[None])


# speedup vs baseline: 1.5243x; 1.0001x over previous
import jax
import jax.numpy as jnp
from jax import lax
from jax.experimental import pallas as pl
from jax.experimental.pallas import tpu as pltpu
from jax.experimental.pallas import tpu_sc as plsc

N_NODES = 10000
D_FEAT = 128
TOPK = 10
BR = 200

NC = 2
NS = 16
NW = NC * NS
CHUNK = 128
N_ACC = 10112
DUMP = 10000
E_EDGE = 160000
EDGE_CHUNKS32 = 80
EDGE_CHUNKS128 = 160
E_PAD_G = NS * EDGE_CHUNKS32 * CHUNK
NSK = N_NODES * TOPK
CORR_CHUNKS = 26
NSK_PAD = NW * CORR_CHUNKS * CHUNK
ROWS_OUT = N_ACC // NS


def _edge_agg_sc(d_feat, n_chunks, ck, serial=False):
    mesh = plsc.VectorSubcoreMesh(core_axis_name="c", subcore_axis_name="s")
    epw = n_chunks * ck

    def body(xm_s, eam_s, src_s, dst_s, xm_t, eam_t, src_t, dst_t, zeros_hbm,
             out_s, out_t,
             src_v0, dst_v0, rows_v0, src_v1, dst_v1, rows_v1,
             eam_v, acc, semL0, semL1, semG0, semG1, semE):
        cid = lax.axis_index("c")
        sid = lax.axis_index("s")
        zr = N_ACC // NS
        pltpu.sync_copy(zeros_hbm.at[pl.ds(sid * zr, zr)],
                        acc.at[pl.ds(sid * zr, zr)])
        plsc.subcore_barrier()
        bufs = ((src_v0, dst_v0, rows_v0, semL0, semG0),
                (src_v1, dst_v1, rows_v1, semL1, semG1))

        def run_graph(xm, eam, src, dst, out):
            def start_loads(b, k):
                src_v, dst_v, rows_v, semL, _ = b
                base = sid * epw + k * ck
                pltpu.async_copy(src.at[pl.ds(base, ck)], src_v, semL)
                pltpu.async_copy(dst.at[pl.ds(base, ck)], dst_v, semL)

            def start_eam(k):
                base = sid * epw + k * ck
                pltpu.async_copy(eam.at[pl.ds(base, ck)], eam_v, semE)

            def launch_gather(b):
                src_v, dst_v, rows_v, semL, semG = b
                pltpu.make_async_copy(src.at[pl.ds(0, ck)], src_v,
                                      semL).wait()
                pltpu.make_async_copy(dst.at[pl.ds(0, ck)], dst_v,
                                      semL).wait()
                pltpu.async_copy(xm.at[src_v], rows_v, semG)

            def finish(b, k):
                src_v, dst_v, rows_v, semL, semG = b
                pltpu.make_async_copy(xm.at[src_v], rows_v, semG).wait()
                pltpu.make_async_copy(eam.at[pl.ds(0, ck)], eam_v,
                                      semE).wait()

                def row_body(i, _):
                    for j in range(d_feat // 16):
                        sl = pl.ds(j * 16, 16)
                        rows_v[i, sl] = jnp.maximum(
                            rows_v[i, sl] + eam_v[i, sl], 0.0)
                    return ()
                lax.fori_loop(0, ck, row_body, (), unroll=4)

                @pl.when(k + 1 < n_chunks)
                def _():
                    start_eam(k + 1)
                pltpu.sync_copy(rows_v, acc.at[dst_v], add=True)

            def phase(this, nxt, k):
                @pl.when(k + 1 < n_chunks)
                def _():
                    launch_gather(nxt)
                finish(this, k)

                @pl.when(k + 2 < n_chunks)
                def _():
                    start_loads(this, k + 2)

            if serial:
                def chunk_body(k, _):
                    base = sid * epw + k * ck
                    pltpu.sync_copy(src.at[pl.ds(base, ck)], src_v0)
                    pltpu.sync_copy(dst.at[pl.ds(base, ck)], dst_v0)
                    gth = pltpu.async_copy(xm.at[src_v0], rows_v0, semG0)
                    pltpu.sync_copy(eam.at[pl.ds(base, ck)], eam_v)
                    gth.wait()

                    def row_body(i, _):
                        for j in range(d_feat // 16):
                            sl = pl.ds(j * 16, 16)
                            rows_v0[i, sl] = jnp.maximum(
                                rows_v0[i, sl] + eam_v[i, sl], 0.0)
                        return ()
                    lax.fori_loop(0, ck, row_body, (), unroll=False)
                    pltpu.sync_copy(rows_v0, acc.at[dst_v0], add=True)
                    return ()
                lax.fori_loop(0, n_chunks, chunk_body, (), unroll=False)
            else:
                start_loads(bufs[0], 0)
                start_loads(bufs[1], 1)
                start_eam(0)
                launch_gather(bufs[0])

                def pair_body(g2, _):
                    phase(bufs[0], bufs[1], 2 * g2)
                    phase(bufs[1], bufs[0], 2 * g2 + 1)
                    return ()
                lax.fori_loop(0, n_chunks // 2, pair_body, (), unroll=False)
            plsc.subcore_barrier()
            pltpu.sync_copy(acc.at[pl.ds(sid * ROWS_OUT, ROWS_OUT)],
                            out.at[pl.ds(sid * ROWS_OUT, ROWS_OUT)])

        @pl.when(cid == 0)
        def _():
            run_graph(xm_s, eam_s, src_s, dst_s, out_s)

        @pl.when(cid == 1)
        def _():
            run_graph(xm_t, eam_t, src_t, dst_t, out_t)

    return pl.kernel(
        body,
        out_type=[
            jax.ShapeDtypeStruct((N_ACC, d_feat), jnp.float32),
            jax.ShapeDtypeStruct((N_ACC, d_feat), jnp.float32),
        ],
        mesh=mesh,
        compiler_params=pltpu.CompilerParams(use_tc_tiling_on_sc=False),
        scratch_types=[
            pltpu.VMEM((ck,), jnp.int32),
            pltpu.VMEM((ck,), jnp.int32),
            pltpu.VMEM((ck, d_feat), jnp.float32),
            pltpu.VMEM((ck,), jnp.int32),
            pltpu.VMEM((ck,), jnp.int32),
            pltpu.VMEM((ck, d_feat), jnp.float32),
            pltpu.VMEM((ck, d_feat), jnp.float32),
            pltpu.VMEM_SHARED((N_ACC, d_feat), jnp.float32),
            pltpu.SemaphoreType.DMA,
            pltpu.SemaphoreType.DMA,
            pltpu.SemaphoreType.DMA,
            pltpu.SemaphoreType.DMA,
            pltpu.SemaphoreType.DMA,
        ],
    )


def _scatter_sum_sc(d_feat, n_chunks):
    mesh = plsc.VectorSubcoreMesh(core_axis_name="c", subcore_axis_name="s")
    rpw = n_chunks * CHUNK

    def body(vals, idx, zeros_hbm, out, vals_v0, idx_v0, vals_v1, idx_v1,
             acc, semL0, semL1):
        cid = lax.axis_index("c")
        sid = lax.axis_index("s")
        wid = cid * NS + sid
        zr = N_ACC // NS
        pltpu.sync_copy(zeros_hbm.at[pl.ds(sid * zr, zr)],
                        acc.at[pl.ds(sid * zr, zr)])
        plsc.subcore_barrier()
        bufs = ((vals_v0, idx_v0, semL0), (vals_v1, idx_v1, semL1))

        def start_loads(b, k):
            vals_v, idx_v, semL = b
            base = wid * rpw + k * CHUNK
            pltpu.async_copy(idx.at[pl.ds(base, CHUNK)], idx_v, semL)
            pltpu.async_copy(vals.at[pl.ds(base, CHUNK)], vals_v, semL)

        def phase(this, nxt, k):
            vals_v, idx_v, semL = this

            @pl.when(k + 1 < n_chunks)
            def _():
                start_loads(nxt, k + 1)
            pltpu.make_async_copy(idx.at[pl.ds(0, CHUNK)], idx_v,
                                  semL).wait()
            pltpu.make_async_copy(vals.at[pl.ds(0, CHUNK)], vals_v,
                                  semL).wait()
            pltpu.sync_copy(vals_v, acc.at[idx_v], add=True)

        start_loads(bufs[0], 0)

        def pair_body(g2, _):
            phase(bufs[0], bufs[1], 2 * g2)
            phase(bufs[1], bufs[0], 2 * g2 + 1)
            return ()
        lax.fori_loop(0, n_chunks // 2, pair_body, (), unroll=False)
        plsc.subcore_barrier()
        pltpu.sync_copy(acc.at[pl.ds(sid * ROWS_OUT, ROWS_OUT)],
                        out.at[cid, pl.ds(sid * ROWS_OUT, ROWS_OUT)])

    return pl.kernel(
        body,
        out_type=jax.ShapeDtypeStruct((NC, N_ACC, d_feat), jnp.float32),
        mesh=mesh,
        compiler_params=pltpu.CompilerParams(use_tc_tiling_on_sc=False),
        scratch_types=[
            pltpu.VMEM((CHUNK, d_feat), jnp.float32),
            pltpu.VMEM((CHUNK,), jnp.int32),
            pltpu.VMEM((CHUNK, d_feat), jnp.float32),
            pltpu.VMEM((CHUNK,), jnp.int32),
            pltpu.VMEM_SHARED((N_ACC, d_feat), jnp.float32),
            pltpu.SemaphoreType.DMA,
            pltpu.SemaphoreType.DMA,
        ],
    )


def _gather_rows_sc(d_feat, n_chunks, n_rows_out):
    mesh = plsc.VectorSubcoreMesh(core_axis_name="c", subcore_axis_name="s")
    rpw = n_chunks * CHUNK

    def body(table, idx, out, idx_v0, rows_v0, idx_v1, rows_v1,
             semL0, semL1, semG0, semG1):
        cid = lax.axis_index("c")
        sid = lax.axis_index("s")
        wid = cid * NS + sid
        bufs = ((idx_v0, rows_v0, semL0, semG0),
                (idx_v1, rows_v1, semL1, semG1))

        def start_load(b, k):
            idx_v, _, semL, _ = b
            base = wid * rpw + k * CHUNK
            pltpu.async_copy(idx.at[pl.ds(base, CHUNK)], idx_v, semL)

        def launch_gather(b):
            idx_v, rows_v, semL, semG = b
            pltpu.make_async_copy(idx.at[pl.ds(0, CHUNK)], idx_v,
                                  semL).wait()
            pltpu.async_copy(table.at[idx_v], rows_v, semG)

        def phase(this, nxt, k):
            idx_v, rows_v, semL, semG = this

            @pl.when(k + 1 < n_chunks)
            def _():
                launch_gather(nxt)
            pltpu.make_async_copy(table.at[idx_v], rows_v, semG).wait()
            base = wid * rpw + k * CHUNK
            pltpu.sync_copy(rows_v, out.at[pl.ds(base, CHUNK)])

            @pl.when(k + 2 < n_chunks)
            def _():
                start_load(this, k + 2)

        start_load(bufs[0], 0)
        start_load(bufs[1], 1)
        launch_gather(bufs[0])

        def pair_body(g2, _):
            phase(bufs[0], bufs[1], 2 * g2)
            phase(bufs[1], bufs[0], 2 * g2 + 1)
            return ()
        lax.fori_loop(0, n_chunks // 2, pair_body, (), unroll=False)

    return pl.kernel(
        body,
        out_type=jax.ShapeDtypeStruct((n_rows_out, d_feat), jnp.float32),
        mesh=mesh,
        compiler_params=pltpu.CompilerParams(use_tc_tiling_on_sc=False),
        scratch_types=[
            pltpu.VMEM((CHUNK,), jnp.int32),
            pltpu.VMEM((CHUNK, d_feat), jnp.float32),
            pltpu.VMEM((CHUNK,), jnp.int32),
            pltpu.VMEM((CHUNK, d_feat), jnp.float32),
            pltpu.SemaphoreType.DMA,
            pltpu.SemaphoreType.DMA,
            pltpu.SemaphoreType.DMA,
            pltpu.SemaphoreType.DMA,
        ],
    )


def _simtopk_body(hs_ref, ht_ref, val_ref, idx_ref):
    sim = lax.dot_general(
        hs_ref[...], ht_ref[...],
        dimension_numbers=(((1,), (1,)), ((), ())),
        preferred_element_type=jnp.float32,
    )
    colid = lax.broadcasted_iota(jnp.int32, sim.shape, 1).astype(jnp.float32)
    neg_inf = jnp.float32(-jnp.inf)
    big = jnp.float32(3e7)
    vals = []
    idxs = []
    for _ in range(TOPK):
        m = jnp.max(sim, axis=1, keepdims=True)
        amax = jnp.min(jnp.where(sim >= m, colid, big), axis=1,
                       keepdims=True)
        vals.append(m)
        idxs.append(amax)
        sim = jnp.where(colid == amax, neg_inf, sim)
    val_ref[...] = jnp.concatenate(vals, axis=1)
    idx_ref[...] = jnp.concatenate(
        [i.astype(jnp.int32) for i in idxs], axis=1)


@jax.jit
def _simtopk(h_s, h_t):
    n_s = h_s.shape[0]
    grid = n_s // BR
    return pl.pallas_call(
        _simtopk_body,
        grid=(grid,),
        in_specs=[
            pl.BlockSpec((BR, D_FEAT), lambda i: (i, 0)),
            pl.BlockSpec((h_t.shape[0], D_FEAT), lambda i: (0, 0)),
        ],
        out_specs=[
            pl.BlockSpec((BR, TOPK), lambda i: (i, 0)),
            pl.BlockSpec((BR, TOPK), lambda i: (i, 0)),
        ],
        out_shape=[
            jax.ShapeDtypeStruct((n_s, TOPK), jnp.float32),
            jax.ShapeDtypeStruct((n_s, TOPK), jnp.int32),
        ],
    )(h_s, h_t)


def _pad_edges(edge_index, edge_attr):
    e = edge_index.shape[1]
    pad = E_PAD_G - e
    src = jnp.pad(edge_index[0], (0, pad))
    dst = jnp.pad(edge_index[1], (0, pad), constant_values=DUMP)
    ea = jnp.pad(edge_attr, ((0, pad), (0, 0)))
    return src, dst, ea


def kernel(x_s, edge_index_s, edge_attr_s, batch_s, x_t, edge_index_t,
           edge_attr_t, batch_t, W1r, W1m, W1e, b1, W2r, W2m, W2e, b2,
           M1, mb1, M2, mb2):
    n_s = x_s.shape[0]
    n_t = x_t.shape[0]
    src_s, dst_s, ea_s = _pad_edges(edge_index_s, edge_attr_s)
    src_t, dst_t, ea_t = _pad_edges(edge_index_t, edge_attr_t)
    zeros128 = jnp.zeros((N_ACC, 128), jnp.float32)
    zeros32 = jnp.zeros((N_ACC, 32), jnp.float32)

    edge_agg128 = _edge_agg_sc(128, 79, 128, serial=True)
    edge_agg32 = _edge_agg_sc(32, EDGE_CHUNKS32, 128)
    scatter32 = _scatter_sum_sc(32, CORR_CHUNKS)
    gather32 = _gather_rows_sc(32, CORR_CHUNKS, NSK_PAD)

    agg_s, agg_t = edge_agg128(x_s @ W1m, ea_s @ W1e, src_s, dst_s,
                               x_t @ W1m, ea_t @ W1e, src_t, dst_t, zeros128)
    h_s = jax.nn.relu(x_s @ W1r + agg_s[:N_NODES] + b1)
    h_t = jax.nn.relu(x_t @ W1r + agg_t[:N_NODES] + b1)

    S_hat, s_idx = _simtopk(h_s, h_t)
    S_0 = jax.nn.softmax(S_hat, axis=-1)

    rng = jax.random.key(12345)
    eam2_s = ea_s @ W2e
    eam2_t = ea_t @ W2e
    flat_idx = s_idx.reshape(-1)
    idx_pad = jnp.pad(flat_idx, (0, NSK_PAD - NSK), constant_values=DUMP)
    idx_pad0 = jnp.pad(flat_idx, (0, NSK_PAD - NSK))
    for step in range(2):
        S = jax.nn.softmax(S_hat, axis=-1)
        r_s = jax.random.normal(jax.random.fold_in(rng, step), (n_s, 32),
                                jnp.float32)
        tmp = (r_s[:, None, :] * S[:, :, None]).reshape(-1, 32)
        tmp = jnp.pad(tmp, ((0, NSK_PAD - NSK), (0, 0)))
        parts = scatter32(tmp, idx_pad, zeros32)
        r_t = parts[0, :N_NODES] + parts[1, :N_NODES]

        agg2_s, agg2_t = edge_agg32(r_s @ W2m, eam2_s, src_s, dst_s,
                                    r_t @ W2m, eam2_t, src_t, dst_t, zeros32)
        o_s = jax.nn.relu(r_s @ W2r + agg2_s[:N_NODES] + b2)
        o_t = jax.nn.relu(r_t @ W2r + agg2_t[:N_NODES] + b2)
        ot_g = gather32(o_t, idx_pad0)[:NSK].reshape(n_s, TOPK, 32)
        D = o_s[:, None, :] - ot_g
        upd = (jax.nn.relu(D @ M1 + mb1) @ M2 + mb2)[..., 0]
        S_hat = S_hat + upd
    S_L = jax.nn.softmax(S_hat, axis=-1)
    return (S_0[None], S_L[None], s_idx[None])
